# Initial kernel scaffold; baseline (speedup 1.0000x reference)
import jax
import jax.numpy as jnp
from jax.experimental import pallas as pl


def kernel(hidden_states, gate_w, W1, b1, W2, b2, sW1, sb1, sW2, sb2):
    raise NotImplementedError("write your pallas kernel here")


# speedup vs baseline: 2.0537x; 2.0537x over previous
import functools

import jax
import jax.numpy as jnp
from jax.experimental import pallas as pl
from jax.experimental.pallas import tpu as pltpu

DIM = 768
INNER = 3072
E = 8
TOPK = 2
TILE = 512


def _moe_step(x_ref, gw_ref, w1_ref, b1_ref, w2_ref, b2_ref, out_ref):
    j = pl.program_id(1)
    x = x_ref[...]
    xb = x.astype(jnp.bfloat16)

    logits = jnp.dot(xb, gw_ref[...].T, preferred_element_type=jnp.float32)
    scores = jax.nn.softmax(logits, axis=-1)
    s_i = scores[:, :, None]
    s_k = scores[:, None, :]
    idx = jax.lax.broadcasted_iota(jnp.int32, (1, E, E), 1)
    kdx = jax.lax.broadcasted_iota(jnp.int32, (1, E, E), 2)
    beats = (s_k > s_i) | ((s_k == s_i) & (kdx < idx))
    rank = jnp.sum(beats.astype(jnp.int32), axis=2)
    combine = jnp.where(rank < TOPK, scores, 0.0)
    combine9 = jnp.concatenate(
        [combine, jnp.ones((combine.shape[0], 1), jnp.float32)], axis=1)
    cols = jax.lax.broadcasted_iota(jnp.int32, (1, E + 1), 1)
    scale = jnp.sum(jnp.where(cols == j, combine9, 0.0), axis=1, keepdims=True)

    h = jnp.dot(xb, w1_ref[0], preferred_element_type=jnp.float32)
    h = h + b1_ref[0]
    h = 0.5 * h * (1.0 + jax.lax.erf(h * 0.7071067811865476))
    y = jnp.dot(h.astype(jnp.bfloat16), w2_ref[0],
                preferred_element_type=jnp.float32)
    y = y + b2_ref[0]
    contrib = scale * y

    @pl.when(j == 0)
    def _init():
        out_ref[...] = contrib

    @pl.when(j != 0)
    def _acc():
        out_ref[...] = out_ref[...] + contrib


@functools.partial(jax.jit, static_argnames=())
def _moe_block(x, gate_w, W1a, b1a, W2a, b2a):
    T = x.shape[0]
    grid = (T // TILE, E + 1)
    return pl.pallas_call(
        _moe_step,
        grid=grid,
        in_specs=[
            pl.BlockSpec((TILE, DIM), lambda i, j: (i, 0)),
            pl.BlockSpec((E, DIM), lambda i, j: (0, 0)),
            pl.BlockSpec((1, DIM, INNER), lambda i, j: (j, 0, 0)),
            pl.BlockSpec((1, 1, INNER), lambda i, j: (j, 0, 0)),
            pl.BlockSpec((1, INNER, DIM), lambda i, j: (j, 0, 0)),
            pl.BlockSpec((1, 1, DIM), lambda i, j: (j, 0, 0)),
        ],
        out_specs=pl.BlockSpec((TILE, DIM), lambda i, j: (i, 0)),
        out_shape=jax.ShapeDtypeStruct((T, DIM), jnp.float32),
        compiler_params=pltpu.CompilerParams(
            dimension_semantics=("parallel", "arbitrary")),
    )(x, gate_w, W1a, b1a, W2a, b2a)


def kernel(hidden_states, gate_w, W1, b1, W2, b2, sW1, sb1, sW2, sb2):
    orig_shape = hidden_states.shape
    x = hidden_states.reshape(-1, orig_shape[-1])
    W1a = jnp.concatenate([W1, sW1[None]], axis=0).astype(jnp.bfloat16)
    W2a = jnp.concatenate([W2, sW2[None]], axis=0).astype(jnp.bfloat16)
    b1a = jnp.concatenate([b1, sb1[None]], axis=0).reshape(E + 1, 1, INNER)
    b2a = jnp.concatenate([b2, sb2[None]], axis=0).reshape(E + 1, 1, DIM)
    gwb = gate_w.astype(jnp.bfloat16)
    y = _moe_block(x, gwb, W1a, b1a, W2a, b2a)
    return y.reshape(orig_shape)


# speedup vs baseline: 2.7439x; 1.3361x over previous
import functools

import jax
import jax.numpy as jnp
from jax.experimental import pallas as pl
from jax.experimental.pallas import tpu as pltpu
from jax.experimental.pallas import tpu_sc as plsc

DIM = 768
INNER = 3072
E = 8
TOPK = 2
T = 2048
S = T * TOPK
TILE_G = 128
NSTEP = (S // 2) // TILE_G + E - 1

@functools.cache
def _vector_mesh():
    return plsc.VectorSubcoreMesh(
        core_axis_name="core", subcore_axis_name="subcore")


def _gating_kernel(x_ref, gw_ref, r_ref, w_ref, offs_ref):
    xb = x_ref[...].astype(jnp.bfloat16)
    logits = jnp.dot(xb, gw_ref[...].T, preferred_element_type=jnp.float32)
    scores = jax.nn.softmax(logits, axis=-1)
    s_i = scores[:, :, None]
    s_k = scores[:, None, :]
    idx = jax.lax.broadcasted_iota(jnp.int32, (1, E, E), 1)
    kdx = jax.lax.broadcasted_iota(jnp.int32, (1, E, E), 2)
    beats = (s_k > s_i) | ((s_k == s_i) & (kdx < idx))
    rank = jnp.sum(beats.astype(jnp.int32), axis=2)
    oh0 = (rank == 0)
    oh1 = (rank == 1)
    w0 = jnp.sum(jnp.where(oh0, scores, 0.0), axis=1, keepdims=True)
    w1 = jnp.sum(jnp.where(oh1, scores, 0.0), axis=1, keepdims=True)
    w_ref[...] = jnp.concatenate([w0, w1], axis=1)

    rank_t = jnp.transpose(rank)
    oh_t = jnp.concatenate(
        [(rank_t == 0), (rank_t == 1)], axis=1).astype(jnp.int32)
    cum = oh_t
    shift = 1
    while shift < S:
        cum = cum + jnp.concatenate(
            [jnp.zeros((E, shift), jnp.int32), cum[:, :-shift]], axis=1)
        shift *= 2
    pos = jnp.sum(oh_t * (cum - 1), axis=0, keepdims=True)
    counts = cum[:, -1:]
    incl = counts
    shift = 1
    while shift < E:
        incl = incl + jnp.concatenate(
            [jnp.zeros((shift, 1), jnp.int32), incl[:-shift]], axis=0)
        shift *= 2
    offs_excl = incl - counts
    base = jnp.sum(oh_t * offs_excl, axis=0, keepdims=True)
    r_ref[...] = base + pos
    offs_ref[...] = jnp.concatenate(
        [jnp.zeros((1, 1), jnp.int32),
         jnp.transpose(incl),
         jnp.zeros((1, 16 - E - 1), jnp.int32)], axis=1)


def _gating(x, gwb):
    return pl.pallas_call(
        _gating_kernel,
        grid=(1,),
        in_specs=[
            pl.BlockSpec((T, DIM), lambda i: (0, 0)),
            pl.BlockSpec((E, DIM), lambda i: (0, 0)),
        ],
        out_specs=[
            pl.BlockSpec((1, S), lambda i: (0, 0)),
            pl.BlockSpec((T, TOPK), lambda i: (0, 0)),
            pl.BlockSpec((1, 16), lambda i: (0, 0)),
        ],
        out_shape=[
            jax.ShapeDtypeStruct((1, S), jnp.int32),
            jax.ShapeDtypeStruct((T, TOPK), jnp.float32),
            jax.ShapeDtypeStruct((1, 16), jnp.int32),
        ],
    )(x, gwb)


_NWORK = 32


def _sc_scatter(x, r1d):
    bpw = S // _NWORK

    @functools.partial(
        pl.kernel, mesh=_vector_mesh(),
        out_type=jax.ShapeDtypeStruct((S, DIM), jnp.float32),
        scratch_types=[
            pltpu.VMEM((bpw,), jnp.int32),
            pltpu.VMEM((bpw, DIM), jnp.float32),
            pltpu.SemaphoreType.DMA,
        ],
    )
    def kern(x_hbm, r_hbm, xs_hbm, idx_v, rows_v, sem):
        wid = (jax.lax.axis_index("subcore") * 2
               + jax.lax.axis_index("core"))
        base = wid * bpw
        src = jax.lax.rem(base, T)
        pltpu.sync_copy(r_hbm.at[pl.ds(base, bpw)], idx_v)
        pltpu.sync_copy(x_hbm.at[pl.ds(src, bpw)], rows_v)
        pltpu.async_copy(rows_v, xs_hbm.at[idx_v], sem).wait()

    return kern(x, r1d)


def _grouped_kernel(meta_ref, xs_ref, w1_ref, b1_ref, w2_ref, b2_ref,
                    out_ref):
    c = pl.program_id(0)
    m = pl.program_id(1)
    tile = meta_ref[c, m, 0]
    start = meta_ref[c, m, 2]
    end = meta_ref[c, m, 3]
    init = meta_ref[c, m, 4]

    @pl.when(start < end)
    def _():
        xb = xs_ref[...].astype(jnp.bfloat16)
        h = jnp.dot(xb, w1_ref[0], preferred_element_type=jnp.float32)
        h = h + b1_ref[0]
        h = 0.5 * h * (1.0 + jax.lax.erf(h * 0.7071067811865476))
        y = jnp.dot(h.astype(jnp.bfloat16), w2_ref[0],
                    preferred_element_type=jnp.float32)
        y = y + b2_ref[0]
        rows = tile * TILE_G + jax.lax.broadcasted_iota(
            jnp.int32, (TILE_G, 1), 0)
        mask = (rows >= start) & (rows < end)
        prev = jnp.where(init == 1, jnp.zeros_like(y), out_ref[...])
        out_ref[...] = jnp.where(mask, y, prev)


def _grouped(meta, xs, W1b, b1r, W2b, b2r):
    grid_spec = pltpu.PrefetchScalarGridSpec(
        num_scalar_prefetch=1,
        grid=(2, NSTEP),
        in_specs=[
            pl.BlockSpec((TILE_G, DIM), lambda c, m, meta: (meta[c, m, 0], 0)),
            pl.BlockSpec((1, DIM, INNER),
                         lambda c, m, meta: (meta[c, m, 1], 0, 0)),
            pl.BlockSpec((1, 1, INNER),
                         lambda c, m, meta: (meta[c, m, 1], 0, 0)),
            pl.BlockSpec((1, INNER, DIM),
                         lambda c, m, meta: (meta[c, m, 1], 0, 0)),
            pl.BlockSpec((1, 1, DIM),
                         lambda c, m, meta: (meta[c, m, 1], 0, 0)),
        ],
        out_specs=pl.BlockSpec((TILE_G, DIM),
                               lambda c, m, meta: (meta[c, m, 0], 0)),
    )
    return pl.pallas_call(
        _grouped_kernel,
        grid_spec=grid_spec,
        out_shape=jax.ShapeDtypeStruct((S, DIM), jnp.float32),
        compiler_params=pltpu.CompilerParams(
            dimension_semantics=("parallel", "arbitrary")),
    )(meta, xs, W1b, b1r, W2b, b2r)


def _make_meta(offs9, c):
    base = c * (S // 2)
    half = S // 2
    ntile = half // TILE_G
    tb = base + jnp.arange(ntile + 1, dtype=jnp.int32) * TILE_G
    gb = jnp.clip(offs9[1:E], base, base + half)
    cuts = jnp.sort(jnp.concatenate([tb, gb]))
    start = cuts[:-1]
    end = cuts[1:]
    tile = jnp.minimum(start // TILE_G, (base + half) // TILE_G - 1)
    group = jnp.clip(jnp.searchsorted(offs9, start, side="right") - 1, 0, E - 1)
    group = group.astype(jnp.int32)
    nonempty = start < end
    n = start.shape[0]
    ar = jnp.arange(n)
    earlier = ((tile[None, :] == tile[:, None]) & nonempty[None, :]
               & (ar[None, :] < ar[:, None]))
    init = nonempty & ~jnp.any(earlier, axis=1)
    return jnp.stack([tile, group, start, end, init.astype(jnp.int32)], 1)


def _shared_kernel(x_ref, w1_ref, b1_ref, w2_ref, b2_ref, out_ref):
    xb = x_ref[...].astype(jnp.bfloat16)
    h = jnp.dot(xb, w1_ref[...], preferred_element_type=jnp.float32)
    h = h + b1_ref[...]
    h = 0.5 * h * (1.0 + jax.lax.erf(h * 0.7071067811865476))
    y = jnp.dot(h.astype(jnp.bfloat16), w2_ref[...],
                preferred_element_type=jnp.float32)
    out_ref[...] = y + b2_ref[...]


def _shared(x, sW1b, sb1r, sW2b, sb2r):
    tile = 512
    return pl.pallas_call(
        _shared_kernel,
        grid=(T // tile,),
        in_specs=[
            pl.BlockSpec((tile, DIM), lambda i: (i, 0)),
            pl.BlockSpec((DIM, INNER), lambda i: (0, 0)),
            pl.BlockSpec((1, INNER), lambda i: (0, 0)),
            pl.BlockSpec((INNER, DIM), lambda i: (0, 0)),
            pl.BlockSpec((1, DIM), lambda i: (0, 0)),
        ],
        out_specs=pl.BlockSpec((tile, DIM), lambda i: (i, 0)),
        out_shape=jax.ShapeDtypeStruct((T, DIM), jnp.float32),
        compiler_params=pltpu.CompilerParams(
            dimension_semantics=("parallel",)),
    )(x, sW1b, sb1r, sW2b, sb2r)


def _sc_gather(ys, r1d):
    bpw = T // _NWORK
    out_t = jax.ShapeDtypeStruct((T, DIM), jnp.float32)

    @functools.partial(
        pl.kernel, mesh=_vector_mesh(),
        out_type=(out_t, out_t),
        scratch_types=[
            pltpu.VMEM((bpw,), jnp.int32),
            pltpu.VMEM((bpw, DIM), jnp.float32),
            pltpu.SemaphoreType.DMA,
        ],
    )
    def kern(ys_hbm, r_hbm, g0_hbm, g1_hbm, idx_v, rows_v, sem):
        wid = (jax.lax.axis_index("subcore") * 2
               + jax.lax.axis_index("core"))
        base = wid * bpw
        pltpu.sync_copy(r_hbm.at[pl.ds(base, bpw)], idx_v)
        pltpu.async_copy(ys_hbm.at[idx_v], rows_v, sem).wait()
        pltpu.sync_copy(rows_v, g0_hbm.at[pl.ds(base, bpw)])
        pltpu.sync_copy(r_hbm.at[pl.ds(T + base, bpw)], idx_v)
        pltpu.async_copy(ys_hbm.at[idx_v], rows_v, sem).wait()
        pltpu.sync_copy(rows_v, g1_hbm.at[pl.ds(base, bpw)])

    return kern(ys, r1d)


def _combine_kernel(sh_ref, g0_ref, g1_ref, w_ref, out_ref):
    w0 = w_ref[:, 0:1]
    w1 = w_ref[:, 1:2]
    out_ref[...] = sh_ref[...] + w0 * g0_ref[...] + w1 * g1_ref[...]


def _combine(sh, g0, g1, w):
    tile = 512
    return pl.pallas_call(
        _combine_kernel,
        grid=(T // tile,),
        in_specs=[
            pl.BlockSpec((tile, DIM), lambda i: (i, 0)),
            pl.BlockSpec((tile, DIM), lambda i: (i, 0)),
            pl.BlockSpec((tile, DIM), lambda i: (i, 0)),
            pl.BlockSpec((tile, TOPK), lambda i: (i, 0)),
        ],
        out_specs=pl.BlockSpec((tile, DIM), lambda i: (i, 0)),
        out_shape=jax.ShapeDtypeStruct((T, DIM), jnp.float32),
        compiler_params=pltpu.CompilerParams(
            dimension_semantics=("parallel",)),
    )(sh, g0, g1, w)


def kernel(hidden_states, gate_w, W1, b1, W2, b2, sW1, sb1, sW2, sb2):
    orig_shape = hidden_states.shape
    x = hidden_states.reshape(-1, orig_shape[-1])

    W1b = W1.astype(jnp.bfloat16)
    W2b = W2.astype(jnp.bfloat16)
    b1r = b1.reshape(E, 1, INNER)
    b2r = b2.reshape(E, 1, DIM)
    sW1b = sW1.astype(jnp.bfloat16)
    sW2b = sW2.astype(jnp.bfloat16)
    sb1r = sb1.reshape(1, INNER)
    sb2r = sb2.reshape(1, DIM)
    gwb = gate_w.astype(jnp.bfloat16)

    r, w, offs = _gating(x, gwb)
    offs9 = offs[0, :E + 1]
    meta = jnp.stack([_make_meta(offs9, 0), _make_meta(offs9, 1)])

    r1d = r.reshape(S)
    xs = _sc_scatter(x, r1d)
    ys = _grouped(meta, xs, W1b, b1r, W2b, b2r)
    sh = _shared(x, sW1b, sb1r, sW2b, sb2r)
    g0, g1 = _sc_gather(ys, r1d)
    y = _combine(sh, g0, g1, w)
    return y.reshape(orig_shape)


# speedup vs baseline: 3.4368x; 1.2525x over previous
import functools

import jax
import jax.numpy as jnp
from jax.experimental import pallas as pl
from jax.experimental.pallas import tpu as pltpu
from jax.experimental.pallas import tpu_sc as plsc

DIM = 768
INNER = 3072
E = 8
TOPK = 2
T = 2048
S = T * TOPK
TILE_G = 128
NSTEP = (S // 2) // TILE_G + E - 1

@functools.cache
def _vector_mesh():
    return plsc.VectorSubcoreMesh(
        core_axis_name="core", subcore_axis_name="subcore")


def _gating_kernel(x_ref, gw_ref, r_ref, w_ref, offs_ref):
    xb = x_ref[...].astype(jnp.bfloat16)
    logits = jnp.dot(xb, gw_ref[...].T, preferred_element_type=jnp.float32)
    scores = jax.nn.softmax(logits, axis=-1)
    s_i = scores[:, :, None]
    s_k = scores[:, None, :]
    idx = jax.lax.broadcasted_iota(jnp.int32, (1, E, E), 1)
    kdx = jax.lax.broadcasted_iota(jnp.int32, (1, E, E), 2)
    beats = (s_k > s_i) | ((s_k == s_i) & (kdx < idx))
    rank = jnp.sum(beats.astype(jnp.int32), axis=2)
    oh0 = (rank == 0)
    oh1 = (rank == 1)
    w0 = jnp.sum(jnp.where(oh0, scores, 0.0), axis=1, keepdims=True)
    w1 = jnp.sum(jnp.where(oh1, scores, 0.0), axis=1, keepdims=True)
    w_ref[...] = jnp.concatenate([w0, w1], axis=1)

    rank_t = jnp.transpose(rank)
    oh_t = jnp.concatenate(
        [(rank_t == 0), (rank_t == 1)], axis=1).astype(jnp.int32)
    cum = oh_t
    shift = 1
    while shift < S:
        cum = cum + jnp.concatenate(
            [jnp.zeros((E, shift), jnp.int32), cum[:, :-shift]], axis=1)
        shift *= 2
    pos = jnp.sum(oh_t * (cum - 1), axis=0, keepdims=True)
    counts = cum[:, -1:]
    incl = counts
    shift = 1
    while shift < E:
        incl = incl + jnp.concatenate(
            [jnp.zeros((shift, 1), jnp.int32), incl[:-shift]], axis=0)
        shift *= 2
    offs_excl = incl - counts
    base = jnp.sum(oh_t * offs_excl, axis=0, keepdims=True)
    r_ref[...] = base + pos
    offs_ref[...] = jnp.concatenate(
        [jnp.zeros((1, 1), jnp.int32),
         jnp.transpose(incl),
         jnp.zeros((1, 16 - E - 1), jnp.int32)], axis=1)


def _gating(x, gwb):
    return pl.pallas_call(
        _gating_kernel,
        grid=(1,),
        in_specs=[
            pl.BlockSpec((T, DIM), lambda i: (0, 0)),
            pl.BlockSpec((E, DIM), lambda i: (0, 0)),
        ],
        out_specs=[
            pl.BlockSpec((1, S), lambda i: (0, 0)),
            pl.BlockSpec((T, TOPK), lambda i: (0, 0)),
            pl.BlockSpec((1, 16), lambda i: (0, 0)),
        ],
        out_shape=[
            jax.ShapeDtypeStruct((1, S), jnp.int32),
            jax.ShapeDtypeStruct((T, TOPK), jnp.float32),
            jax.ShapeDtypeStruct((1, 16), jnp.int32),
        ],
    )(x, gwb)


_NWORK = 32


def _sc_scatter(x, r1d):
    bpw = S // _NWORK

    @functools.partial(
        pl.kernel, mesh=_vector_mesh(),
        out_type=jax.ShapeDtypeStruct((S, DIM), jnp.float32),
        scratch_types=[
            pltpu.VMEM((bpw,), jnp.int32),
            pltpu.VMEM((bpw, DIM), jnp.float32),
            pltpu.SemaphoreType.DMA,
        ],
    )
    def kern(x_hbm, r_hbm, xs_hbm, idx_v, rows_v, sem):
        wid = (jax.lax.axis_index("subcore") * 2
               + jax.lax.axis_index("core"))
        base = wid * bpw
        src = jax.lax.rem(base, T)
        pltpu.sync_copy(r_hbm.at[pl.ds(base, bpw)], idx_v)
        pltpu.sync_copy(x_hbm.at[pl.ds(src, bpw)], rows_v)
        pltpu.async_copy(rows_v, xs_hbm.at[idx_v], sem).wait()

    return kern(x, r1d)


def _grouped_kernel(meta_ref, xs_ref, w1_ref, b1_ref, w2_ref, b2_ref,
                    out_ref):
    c = pl.program_id(0)
    m = pl.program_id(1)
    tile = meta_ref[c, m, 0]
    start = meta_ref[c, m, 2]
    end = meta_ref[c, m, 3]
    init = meta_ref[c, m, 4]

    @pl.when(start < end)
    def _():
        xb = xs_ref[...].astype(jnp.bfloat16)
        h = jnp.dot(xb, w1_ref[0].astype(jnp.bfloat16),
                    preferred_element_type=jnp.float32)
        h = h + b1_ref[0]
        h = 0.5 * h * (1.0 + jax.lax.erf(h * 0.7071067811865476))
        y = jnp.dot(h.astype(jnp.bfloat16), w2_ref[0].astype(jnp.bfloat16),
                    preferred_element_type=jnp.float32)
        y = y + b2_ref[0]
        rows = tile * TILE_G + jax.lax.broadcasted_iota(
            jnp.int32, (TILE_G, 1), 0)
        mask = (rows >= start) & (rows < end)
        prev = jnp.where(init == 1, jnp.zeros_like(y), out_ref[...])
        out_ref[...] = jnp.where(mask, y, prev)


def _grouped(meta, xs, W1b, b1r, W2b, b2r):
    grid_spec = pltpu.PrefetchScalarGridSpec(
        num_scalar_prefetch=1,
        grid=(2, NSTEP),
        in_specs=[
            pl.BlockSpec((TILE_G, DIM), lambda c, m, meta: (meta[c, m, 0], 0)),
            pl.BlockSpec((1, DIM, INNER),
                         lambda c, m, meta: (meta[c, m, 1], 0, 0)),
            pl.BlockSpec((1, 1, INNER),
                         lambda c, m, meta: (meta[c, m, 1], 0, 0)),
            pl.BlockSpec((1, INNER, DIM),
                         lambda c, m, meta: (meta[c, m, 1], 0, 0)),
            pl.BlockSpec((1, 1, DIM),
                         lambda c, m, meta: (meta[c, m, 1], 0, 0)),
        ],
        out_specs=pl.BlockSpec((TILE_G, DIM),
                               lambda c, m, meta: (meta[c, m, 0], 0)),
    )
    return pl.pallas_call(
        _grouped_kernel,
        grid_spec=grid_spec,
        out_shape=jax.ShapeDtypeStruct((S, DIM), jnp.float32),
        compiler_params=pltpu.CompilerParams(
            dimension_semantics=("parallel", "arbitrary")),
    )(meta, xs, W1b, b1r, W2b, b2r)


def _make_meta(offs9, c):
    base = c * (S // 2)
    half = S // 2
    ntile = half // TILE_G
    tb = base + jnp.arange(ntile + 1, dtype=jnp.int32) * TILE_G
    gb = jnp.clip(offs9[1:E], base, base + half)
    cuts = jnp.sort(jnp.concatenate([tb, gb]))
    start = cuts[:-1]
    end = cuts[1:]
    tile = jnp.minimum(start // TILE_G, (base + half) // TILE_G - 1)
    group = jnp.clip(jnp.searchsorted(offs9, start, side="right") - 1, 0, E - 1)
    group = group.astype(jnp.int32)
    nonempty = start < end
    n = start.shape[0]
    ar = jnp.arange(n)
    earlier = ((tile[None, :] == tile[:, None]) & nonempty[None, :]
               & (ar[None, :] < ar[:, None]))
    init = nonempty & ~jnp.any(earlier, axis=1)
    return jnp.stack([tile, group, start, end, init.astype(jnp.int32)], 1)


def _shared_kernel(x_ref, w1_ref, b1_ref, w2_ref, b2_ref, out_ref):
    xb = x_ref[...].astype(jnp.bfloat16)
    h = jnp.dot(xb, w1_ref[...].astype(jnp.bfloat16),
                preferred_element_type=jnp.float32)
    h = h + b1_ref[...]
    h = 0.5 * h * (1.0 + jax.lax.erf(h * 0.7071067811865476))
    y = jnp.dot(h.astype(jnp.bfloat16), w2_ref[...].astype(jnp.bfloat16),
                preferred_element_type=jnp.float32)
    out_ref[...] = y + b2_ref[...]


def _shared(x, sW1b, sb1r, sW2b, sb2r):
    tile = 512
    return pl.pallas_call(
        _shared_kernel,
        grid=(T // tile,),
        in_specs=[
            pl.BlockSpec((tile, DIM), lambda i: (i, 0)),
            pl.BlockSpec((DIM, INNER), lambda i: (0, 0)),
            pl.BlockSpec((1, INNER), lambda i: (0, 0)),
            pl.BlockSpec((INNER, DIM), lambda i: (0, 0)),
            pl.BlockSpec((1, DIM), lambda i: (0, 0)),
        ],
        out_specs=pl.BlockSpec((tile, DIM), lambda i: (i, 0)),
        out_shape=jax.ShapeDtypeStruct((T, DIM), jnp.float32),
        compiler_params=pltpu.CompilerParams(
            dimension_semantics=("parallel",)),
    )(x, sW1b, sb1r, sW2b, sb2r)


def _sc_gather(ys, r1d):
    bpw = T // _NWORK
    out_t = jax.ShapeDtypeStruct((T, DIM), jnp.float32)

    @functools.partial(
        pl.kernel, mesh=_vector_mesh(),
        out_type=(out_t, out_t),
        scratch_types=[
            pltpu.VMEM((bpw,), jnp.int32),
            pltpu.VMEM((bpw, DIM), jnp.float32),
            pltpu.SemaphoreType.DMA,
        ],
    )
    def kern(ys_hbm, r_hbm, g0_hbm, g1_hbm, idx_v, rows_v, sem):
        wid = (jax.lax.axis_index("subcore") * 2
               + jax.lax.axis_index("core"))
        base = wid * bpw
        pltpu.sync_copy(r_hbm.at[pl.ds(base, bpw)], idx_v)
        pltpu.async_copy(ys_hbm.at[idx_v], rows_v, sem).wait()
        pltpu.sync_copy(rows_v, g0_hbm.at[pl.ds(base, bpw)])
        pltpu.sync_copy(r_hbm.at[pl.ds(T + base, bpw)], idx_v)
        pltpu.async_copy(ys_hbm.at[idx_v], rows_v, sem).wait()
        pltpu.sync_copy(rows_v, g1_hbm.at[pl.ds(base, bpw)])

    return kern(ys, r1d)


def _combine_kernel(sh_ref, g0_ref, g1_ref, w_ref, out_ref):
    w0 = w_ref[:, 0:1]
    w1 = w_ref[:, 1:2]
    out_ref[...] = sh_ref[...] + w0 * g0_ref[...] + w1 * g1_ref[...]


def _combine(sh, g0, g1, w):
    tile = 512
    return pl.pallas_call(
        _combine_kernel,
        grid=(T // tile,),
        in_specs=[
            pl.BlockSpec((tile, DIM), lambda i: (i, 0)),
            pl.BlockSpec((tile, DIM), lambda i: (i, 0)),
            pl.BlockSpec((tile, DIM), lambda i: (i, 0)),
            pl.BlockSpec((tile, TOPK), lambda i: (i, 0)),
        ],
        out_specs=pl.BlockSpec((tile, DIM), lambda i: (i, 0)),
        out_shape=jax.ShapeDtypeStruct((T, DIM), jnp.float32),
        compiler_params=pltpu.CompilerParams(
            dimension_semantics=("parallel",)),
    )(sh, g0, g1, w)


def kernel(hidden_states, gate_w, W1, b1, W2, b2, sW1, sb1, sW2, sb2):
    orig_shape = hidden_states.shape
    x = hidden_states.reshape(-1, orig_shape[-1])

    b1r = b1.reshape(E, 1, INNER)
    b2r = b2.reshape(E, 1, DIM)
    sb1r = sb1.reshape(1, INNER)
    sb2r = sb2.reshape(1, DIM)
    gwb = gate_w.astype(jnp.bfloat16)

    r, w, offs = _gating(x, gwb)
    offs9 = offs[0, :E + 1]
    meta = jnp.stack([_make_meta(offs9, 0), _make_meta(offs9, 1)])

    r1d = r.reshape(S)
    xs = _sc_scatter(x, r1d)
    ys = _grouped(meta, xs, W1, b1r, W2, b2r)
    sh = _shared(x, sW1, sb1r, sW2, sb2r)
    g0, g1 = _sc_gather(ys, r1d)
    y = _combine(sh, g0, g1, w)
    return y.reshape(orig_shape)


# speedup vs baseline: 3.8320x; 1.1150x over previous
import functools

import jax
import jax.numpy as jnp
from jax.experimental import pallas as pl
from jax.experimental.pallas import tpu as pltpu
from jax.experimental.pallas import tpu_sc as plsc

DIM = 768
INNER = 3072
E = 8
TOPK = 2
T = 2048
S = T * TOPK
TILE_G = 128
NSTEP = S // TILE_G + E - 1

@functools.cache
def _vector_mesh():
    return plsc.VectorSubcoreMesh(
        core_axis_name="core", subcore_axis_name="subcore")


def _gating_kernel(x_ref, gw_ref, r_ref, w_ref, meta_ref):
    xb = x_ref[...].astype(jnp.bfloat16)
    gwb = gw_ref[...].astype(jnp.bfloat16)
    logits = jnp.dot(xb, gwb.T, preferred_element_type=jnp.float32)
    lt = jnp.transpose(logits)
    lt = lt - jnp.max(lt, axis=0, keepdims=True)
    el = jnp.exp(lt)
    scores = el / jnp.sum(el, axis=0, keepdims=True)

    eidx = jax.lax.broadcasted_iota(jnp.int32, (E, 1), 0)
    rank = jnp.zeros((E, T), jnp.int32)
    for k in range(E):
        sk = scores[k:k + 1, :]
        beats = (sk > scores) | ((sk == scores) & (k < eidx))
        rank = rank + beats.astype(jnp.int32)
    oh0 = rank == 0
    oh1 = rank == 1
    w0 = jnp.sum(jnp.where(oh0, scores, 0.0), axis=0, keepdims=True)
    w1 = jnp.sum(jnp.where(oh1, scores, 0.0), axis=0, keepdims=True)
    w_ref[...] = jnp.transpose(jnp.concatenate([w0, w1], axis=0))

    oh_t = jnp.concatenate([oh0, oh1], axis=1).astype(jnp.int32)
    cum = oh_t
    shift = 1
    while shift < S:
        cum = cum + jnp.concatenate(
            [jnp.zeros((E, shift), jnp.int32), cum[:, :-shift]], axis=1)
        shift *= 2
    pos = jnp.sum(oh_t * (cum - 1), axis=0, keepdims=True)
    counts = cum[:, -1:]
    incl = counts
    shift = 1
    while shift < E:
        incl = incl + jnp.concatenate(
            [jnp.zeros((shift, 1), jnp.int32), incl[:-shift]], axis=0)
        shift *= 2
    offs_excl = incl - counts
    base = jnp.sum(oh_t * offs_excl, axis=0, keepdims=True)
    r_ref[...] = base + pos

    offs_col = jnp.concatenate([jnp.zeros((1, 1), jnp.int32), incl], axis=0)
    ntile = S // TILE_G
    tb = jax.lax.broadcasted_iota(jnp.int32, (1, ntile + 1), 1) * TILE_G
    gb = jnp.transpose(offs_col[1:E])
    v = jnp.concatenate([tb, gb], axis=1)
    n = v.shape[1]
    v_col = jnp.transpose(v)
    i_col = jax.lax.broadcasted_iota(jnp.int32, (n, 1), 0)
    j_row = jax.lax.broadcasted_iota(jnp.int32, (1, n), 1)
    less = (v < v_col) | ((v == v_col) & (j_row < i_col))
    rnk = jnp.sum(less.astype(jnp.int32), axis=1, keepdims=True)
    onehot = (rnk == j_row).astype(jnp.int32)
    cuts = jnp.sum(onehot * v_col, axis=0, keepdims=True)
    start = cuts[:, :NSTEP]
    end = cuts[:, 1:]
    tile = jnp.minimum(start // TILE_G, ntile - 1)
    grp = jnp.clip(
        jnp.sum((offs_col <= start).astype(jnp.int32), axis=0,
                keepdims=True) - 1, 0, E - 1)
    nonempty = start < end
    tile_col = jnp.transpose(tile)
    ne_col = jnp.transpose(nonempty)
    si = jax.lax.broadcasted_iota(jnp.int32, (NSTEP, 1), 0)
    sj = jax.lax.broadcasted_iota(jnp.int32, (1, NSTEP), 1)
    ear = (tile == tile_col) & nonempty & (sj < si)
    anyear = jnp.sum(ear.astype(jnp.int32), axis=1, keepdims=True) > 0
    init = ne_col & ~anyear
    rows = jnp.concatenate(
        [tile, grp, start, end, jnp.transpose(init.astype(jnp.int32))],
        axis=0)
    meta_ref[...] = jnp.pad(rows, ((0, 3), (0, 48 - NSTEP)))


def _gating(x, gw):
    return pl.pallas_call(
        _gating_kernel,
        grid=(1,),
        in_specs=[
            pl.BlockSpec((T, DIM), lambda i: (0, 0)),
            pl.BlockSpec((E, DIM), lambda i: (0, 0)),
        ],
        out_specs=[
            pl.BlockSpec((1, S), lambda i: (0, 0)),
            pl.BlockSpec((T, TOPK), lambda i: (0, 0)),
            pl.BlockSpec((8, 48), lambda i: (0, 0)),
        ],
        out_shape=[
            jax.ShapeDtypeStruct((1, S), jnp.int32),
            jax.ShapeDtypeStruct((T, TOPK), jnp.float32),
            jax.ShapeDtypeStruct((8, 48), jnp.int32),
        ],
    )(x, gw)


_NWORK = 32


def _sc_scatter(x, r1d):
    bpw = S // _NWORK

    @functools.partial(
        pl.kernel, mesh=_vector_mesh(),
        out_type=jax.ShapeDtypeStruct((S, DIM), jnp.float32),
        scratch_types=[
            pltpu.VMEM((bpw,), jnp.int32),
            pltpu.VMEM((bpw, DIM), jnp.float32),
            pltpu.SemaphoreType.DMA,
        ],
    )
    def kern(x_hbm, r_hbm, xs_hbm, idx_v, rows_v, sem):
        wid = (jax.lax.axis_index("subcore") * 2
               + jax.lax.axis_index("core"))
        base = wid * bpw
        src = jax.lax.rem(base, T)
        pltpu.sync_copy(r_hbm.at[pl.ds(base, bpw)], idx_v)
        pltpu.sync_copy(x_hbm.at[pl.ds(src, bpw)], rows_v)
        pltpu.async_copy(rows_v, xs_hbm.at[idx_v], sem).wait()

    return kern(x, r1d)


def _grouped_kernel(meta_ref, xs_ref, w1_ref, b1_ref, w2_ref, b2_ref,
                    out_ref):
    m = pl.program_id(0)
    tile = meta_ref[0, m]
    start = meta_ref[2, m]
    end = meta_ref[3, m]
    init = meta_ref[4, m]

    @pl.when(start < end)
    def _():
        xb = xs_ref[...].astype(jnp.bfloat16)
        nch = 4
        cw = INNER // nch
        hs = []
        for i in range(nch):
            w1c = w1_ref[0][:, i * cw:(i + 1) * cw].astype(jnp.bfloat16)
            hs.append(jnp.dot(xb, w1c, preferred_element_type=jnp.float32))
        h = jnp.concatenate(hs, axis=1)
        h = h + b1_ref[0]
        h = 0.5 * h * (1.0 + jax.lax.erf(h * 0.7071067811865476))
        hb = h.astype(jnp.bfloat16)
        y = jnp.zeros((TILE_G, DIM), jnp.float32)
        for i in range(nch):
            w2c = w2_ref[0][i * cw:(i + 1) * cw, :].astype(jnp.bfloat16)
            y = y + jnp.dot(hb[:, i * cw:(i + 1) * cw], w2c,
                            preferred_element_type=jnp.float32)
        y = y + b2_ref[0]
        rows = tile * TILE_G + jax.lax.broadcasted_iota(
            jnp.int32, (TILE_G, 1), 0)
        mask = (rows >= start) & (rows < end)
        prev = jnp.where(init == 1, jnp.zeros_like(y), out_ref[...])
        out_ref[...] = jnp.where(mask, y, prev)


def _grouped(meta, xs, W1b, b1r, W2b, b2r):
    grid_spec = pltpu.PrefetchScalarGridSpec(
        num_scalar_prefetch=1,
        grid=(NSTEP,),
        in_specs=[
            pl.BlockSpec((TILE_G, DIM), lambda m, meta: (meta[0, m], 0)),
            pl.BlockSpec((1, DIM, INNER),
                         lambda m, meta: (meta[1, m], 0, 0)),
            pl.BlockSpec((1, 1, INNER),
                         lambda m, meta: (meta[1, m], 0, 0)),
            pl.BlockSpec((1, INNER, DIM),
                         lambda m, meta: (meta[1, m], 0, 0)),
            pl.BlockSpec((1, 1, DIM),
                         lambda m, meta: (meta[1, m], 0, 0)),
        ],
        out_specs=pl.BlockSpec((TILE_G, DIM),
                               lambda m, meta: (meta[0, m], 0)),
    )
    return pl.pallas_call(
        _grouped_kernel,
        grid_spec=grid_spec,
        out_shape=jax.ShapeDtypeStruct((S, DIM), jnp.float32),
        compiler_params=pltpu.CompilerParams(
            dimension_semantics=("arbitrary",)),
    )(meta, xs, W1b, b1r, W2b, b2r)


def _shared_kernel(x_ref, w1_ref, b1_ref, w2_ref, b2_ref, out_ref):
    xb = x_ref[...].astype(jnp.bfloat16)
    nch = 4
    cw = INNER // nch
    hs = []
    for i in range(nch):
        w1c = w1_ref[:, i * cw:(i + 1) * cw].astype(jnp.bfloat16)
        hs.append(jnp.dot(xb, w1c, preferred_element_type=jnp.float32))
    h = jnp.concatenate(hs, axis=1)
    h = h + b1_ref[...]
    h = 0.5 * h * (1.0 + jax.lax.erf(h * 0.7071067811865476))
    hb = h.astype(jnp.bfloat16)
    y = jnp.zeros((x_ref.shape[0], DIM), jnp.float32)
    for i in range(nch):
        w2c = w2_ref[i * cw:(i + 1) * cw, :].astype(jnp.bfloat16)
        y = y + jnp.dot(hb[:, i * cw:(i + 1) * cw], w2c,
                        preferred_element_type=jnp.float32)
    out_ref[...] = y + b2_ref[...]


def _shared(x, sW1b, sb1r, sW2b, sb2r):
    tile = 512
    return pl.pallas_call(
        _shared_kernel,
        grid=(T // tile,),
        in_specs=[
            pl.BlockSpec((tile, DIM), lambda i: (i, 0)),
            pl.BlockSpec((DIM, INNER), lambda i: (0, 0)),
            pl.BlockSpec((1, INNER), lambda i: (0, 0)),
            pl.BlockSpec((INNER, DIM), lambda i: (0, 0)),
            pl.BlockSpec((1, DIM), lambda i: (0, 0)),
        ],
        out_specs=pl.BlockSpec((tile, DIM), lambda i: (i, 0)),
        out_shape=jax.ShapeDtypeStruct((T, DIM), jnp.float32),
        compiler_params=pltpu.CompilerParams(
            dimension_semantics=("parallel",)),
    )(x, sW1b, sb1r, sW2b, sb2r)


def _sc_gather(ys, r1d):
    bpw = T // _NWORK
    out_t = jax.ShapeDtypeStruct((T, DIM), jnp.float32)

    @functools.partial(
        pl.kernel, mesh=_vector_mesh(),
        out_type=(out_t, out_t),
        scratch_types=[
            pltpu.VMEM((bpw,), jnp.int32),
            pltpu.VMEM((bpw, DIM), jnp.float32),
            pltpu.SemaphoreType.DMA,
        ],
    )
    def kern(ys_hbm, r_hbm, g0_hbm, g1_hbm, idx_v, rows_v, sem):
        wid = (jax.lax.axis_index("subcore") * 2
               + jax.lax.axis_index("core"))
        base = wid * bpw
        pltpu.sync_copy(r_hbm.at[pl.ds(base, bpw)], idx_v)
        pltpu.async_copy(ys_hbm.at[idx_v], rows_v, sem).wait()
        pltpu.sync_copy(rows_v, g0_hbm.at[pl.ds(base, bpw)])
        pltpu.sync_copy(r_hbm.at[pl.ds(T + base, bpw)], idx_v)
        pltpu.async_copy(ys_hbm.at[idx_v], rows_v, sem).wait()
        pltpu.sync_copy(rows_v, g1_hbm.at[pl.ds(base, bpw)])

    return kern(ys, r1d)


def _combine_kernel(sh_ref, g0_ref, g1_ref, w_ref, out_ref):
    w0 = w_ref[:, 0:1]
    w1 = w_ref[:, 1:2]
    out_ref[...] = sh_ref[...] + w0 * g0_ref[...] + w1 * g1_ref[...]


def _combine(sh, g0, g1, w):
    tile = 512
    return pl.pallas_call(
        _combine_kernel,
        grid=(T // tile,),
        in_specs=[
            pl.BlockSpec((tile, DIM), lambda i: (i, 0)),
            pl.BlockSpec((tile, DIM), lambda i: (i, 0)),
            pl.BlockSpec((tile, DIM), lambda i: (i, 0)),
            pl.BlockSpec((tile, TOPK), lambda i: (i, 0)),
        ],
        out_specs=pl.BlockSpec((tile, DIM), lambda i: (i, 0)),
        out_shape=jax.ShapeDtypeStruct((T, DIM), jnp.float32),
        compiler_params=pltpu.CompilerParams(
            dimension_semantics=("parallel",)),
    )(sh, g0, g1, w)


def kernel(hidden_states, gate_w, W1, b1, W2, b2, sW1, sb1, sW2, sb2):
    orig_shape = hidden_states.shape
    x = hidden_states.reshape(-1, orig_shape[-1])

    b1r = b1.reshape(E, 1, INNER)
    b2r = b2.reshape(E, 1, DIM)
    sb1r = sb1.reshape(1, INNER)
    sb2r = sb2.reshape(1, DIM)

    r, w, meta = _gating(x, gate_w)
    r1d = r.reshape(S)
    xs = _sc_scatter(x, r1d)
    ys = _grouped(meta, xs, W1, b1r, W2, b2r)
    sh = _shared(x, sW1, sb1r, sW2, sb2r)
    g0, g1 = _sc_gather(ys, r1d)
    y = _combine(sh, g0, g1, w)
    return y.reshape(orig_shape)
